# baseline (speedup 1.0000x reference)
import functools

import jax
import jax.numpy as jnp
from jax import lax
from jax.experimental import pallas as pl
from jax.experimental.pallas import tpu as pltpu
from jax.experimental.pallas import tpu_sc as plsc

B = 4096
F = 26
V = 100000
D = 32
TOT = F * D

NC = 2
NS = 16
NW = NC * NS
ROWS = B * F
RPW = ROWS // NW
CHUNK = 128
NCH = RPW // CHUNK
FIRE = 13


def _gather_body(tab_ref, idx_ref, out_ref, idx_v, rows_v, sem):
    wid = lax.axis_index("s") * NC + lax.axis_index("c")
    base = wid * RPW
    pltpu.sync_copy(idx_ref.at[wid], idx_v)
    for start in range(0, NCH, FIRE):
        descs = []
        for j in range(start, min(start + FIRE, NCH)):
            descs.append(
                pltpu.async_copy(
                    tab_ref.at[idx_v.at[j]],
                    rows_v.at[pl.ds(j * CHUNK, CHUNK)],
                    sem,
                )
            )
        for d in descs:
            d.wait()
    pltpu.sync_copy(rows_v, out_ref.at[pl.ds(base, RPW)])


@functools.cache
def _make_gather():
    return pl.kernel(
        _gather_body,
        out_type=jax.ShapeDtypeStruct((ROWS, D), jnp.float32),
        mesh=plsc.VectorSubcoreMesh(core_axis_name="c", subcore_axis_name="s",
                                    num_cores=NC, num_subcores=NS),
        scratch_types=[
            pltpu.VMEM((NCH, CHUNK), jnp.int32),
            pltpu.VMEM((RPW, D), jnp.float32),
            pltpu.SemaphoreType.DMA,
        ],
        compiler_params=pltpu.CompilerParams(use_tc_tiling_on_sc=False),
    )


def _tower_body(x_ref, g_ref, bb_ref, w1_ref, b1_ref, w2_ref, b2_ref,
                w3_ref, b3_ref, out_ref):
    x = x_ref[...]
    mu = jnp.mean(x, axis=0, keepdims=True)
    xc = x - mu
    var = jnp.mean(xc * xc, axis=0, keepdims=True)
    xn = xc * (g_ref[...] * lax.rsqrt(var + 1e-5)) + bb_ref[...]
    h = jnp.dot(xn, w1_ref[...], preferred_element_type=jnp.float32)
    h = jnp.maximum(h + b1_ref[...], 0.0)
    h = jnp.dot(h, w2_ref[...], preferred_element_type=jnp.float32)
    h = jnp.maximum(h + b2_ref[...], 0.0)
    out = jnp.dot(h, w3_ref[...], preferred_element_type=jnp.float32)
    out_ref[...] = out + b3_ref[...]


def _tower(x, g, bb, w1, b1, w2, b2, w3, b3):
    return pl.pallas_call(
        _tower_body,
        out_shape=jax.ShapeDtypeStruct((B, 128), jnp.float32),
    )(x, g, bb, w1, b1, w2, b2, w3, b3)


def kernel(sparse, tables, bn_gamma, bn_beta, W1, b1, W2, b2, W3, b3):
    flat_idx = sparse + jnp.arange(F, dtype=jnp.int32) * V
    idx = flat_idx.reshape(NW, NCH, CHUNK)
    tab = tables.reshape(F * V, D)
    gathered = _make_gather()(tab, idx)
    x = gathered.reshape(B, TOT)
    return _tower(
        x,
        bn_gamma.reshape(1, TOT),
        bn_beta.reshape(1, TOT),
        W1, b1.reshape(1, 512),
        W2, b2.reshape(1, 256),
        W3, b3.reshape(1, 128),
    )
